# initial kernel scaffold (unmeasured)
import jax
import jax.numpy as jnp
from jax import lax
from jax.experimental import pallas as pl
from jax.experimental.pallas import tpu as pltpu


def kernel(x, W):
    t, d = x.shape
    _, v_shard = W.shape
    v_total = 2 * v_shard

    def body(x_ref, w_ref, out_ref, send_sem, recv_sem):
        my_x = lax.axis_index("x")
        my_y = lax.axis_index("y")
        my_z = lax.axis_index("z")
        partner = (1 - my_x, my_y, my_z)

        barrier = pltpu.get_barrier_semaphore()
        pl.semaphore_signal(
            barrier, inc=1, device_id=partner,
            device_id_type=pl.DeviceIdType.MESH,
        )
        pl.semaphore_wait(barrier, 1)

        logits = jnp.dot(
            x_ref[:, :], w_ref[:, :], preferred_element_type=jnp.float32
        )
        out_ref[:, pl.ds(my_x * v_shard, v_shard)] = logits

        rdma = pltpu.make_async_remote_copy(
            src_ref=out_ref.at[:, pl.ds(my_x * v_shard, v_shard)],
            dst_ref=out_ref.at[:, pl.ds(my_x * v_shard, v_shard)],
            send_sem=send_sem,
            recv_sem=recv_sem,
            device_id=partner,
            device_id_type=pl.DeviceIdType.MESH,
        )
        rdma.start()
        rdma.wait()

        vals = out_ref[:, :]
        m = jnp.max(vals, axis=-1, keepdims=True)
        e = jnp.exp(vals - m)
        s = jnp.sum(e, axis=-1, keepdims=True)
        out_ref[:, :] = e / s

    return pl.pallas_call(
        body,
        out_shape=jax.ShapeDtypeStruct((t, v_total), jnp.float32),
        in_specs=[
            pl.BlockSpec(memory_space=pltpu.VMEM),
            pl.BlockSpec(memory_space=pltpu.VMEM),
        ],
        out_specs=pl.BlockSpec(memory_space=pltpu.VMEM),
        scratch_shapes=[
            pltpu.SemaphoreType.DMA,
            pltpu.SemaphoreType.DMA,
        ],
        compiler_params=pltpu.CompilerParams(collective_id=0),
    )(x, W)


# baseline (device time: 244715 ns/iter reference)
import jax
import jax.numpy as jnp
from jax import lax
from jax.experimental import pallas as pl
from jax.experimental.pallas import tpu as pltpu

W_CHUNK = 1024
SM_CHUNK = 2048


def kernel(x, W):
    t, d = x.shape
    _, v_shard = W.shape
    v_total = 2 * v_shard
    n_wchunks = v_shard // W_CHUNK
    n_smchunks = v_total // SM_CHUNK

    def body(x_ref, w_hbm, out_ref, w_buf, copy_sems, send_sem, recv_sem):
        my_x = lax.axis_index("x")
        my_y = lax.axis_index("y")
        my_z = lax.axis_index("z")
        partner = (1 - my_x, my_y, my_z)

        barrier = pltpu.get_barrier_semaphore()
        pl.semaphore_signal(
            barrier, inc=1, device_id=partner,
            device_id_type=pl.DeviceIdType.MESH,
        )
        pl.semaphore_wait(barrier, 1)

        def w_copy(k, slot):
            return pltpu.make_async_copy(
                w_hbm.at[:, pl.ds(k * W_CHUNK, W_CHUNK)],
                w_buf.at[slot],
                copy_sems.at[slot],
            )

        w_copy(0, 0).start()
        xv = x_ref[:, :]
        for k in range(n_wchunks):
            slot = k % 2
            if k + 1 < n_wchunks:
                w_copy(k + 1, (k + 1) % 2).start()
            w_copy(k, slot).wait()
            logits_k = jnp.dot(
                xv, w_buf[slot], preferred_element_type=jnp.float32
            )
            out_ref[:, pl.ds(my_x * v_shard + k * W_CHUNK, W_CHUNK)] = logits_k

        rdma = pltpu.make_async_remote_copy(
            src_ref=out_ref.at[:, pl.ds(my_x * v_shard, v_shard)],
            dst_ref=out_ref.at[:, pl.ds(my_x * v_shard, v_shard)],
            send_sem=send_sem,
            recv_sem=recv_sem,
            device_id=partner,
            device_id_type=pl.DeviceIdType.MESH,
        )
        rdma.start()
        rdma.wait()

        m = jnp.full((t, 1), -jnp.inf, dtype=jnp.float32)
        for k in range(n_smchunks):
            c = out_ref[:, pl.ds(k * SM_CHUNK, SM_CHUNK)]
            m = jnp.maximum(m, jnp.max(c, axis=-1, keepdims=True))
        s = jnp.zeros((t, 1), dtype=jnp.float32)
        for k in range(n_smchunks):
            e = jnp.exp(out_ref[:, pl.ds(k * SM_CHUNK, SM_CHUNK)] - m)
            out_ref[:, pl.ds(k * SM_CHUNK, SM_CHUNK)] = e
            s = s + jnp.sum(e, axis=-1, keepdims=True)
        inv = 1.0 / s
        for k in range(n_smchunks):
            out_ref[:, pl.ds(k * SM_CHUNK, SM_CHUNK)] = (
                out_ref[:, pl.ds(k * SM_CHUNK, SM_CHUNK)] * inv
            )

    return pl.pallas_call(
        body,
        out_shape=jax.ShapeDtypeStruct((t, v_total), jnp.float32),
        in_specs=[
            pl.BlockSpec(memory_space=pltpu.VMEM),
            pl.BlockSpec(memory_space=pl.ANY),
        ],
        out_specs=pl.BlockSpec(memory_space=pltpu.VMEM),
        scratch_shapes=[
            pltpu.VMEM((2, d, W_CHUNK), jnp.float32),
            pltpu.SemaphoreType.DMA((2,)),
            pltpu.SemaphoreType.DMA,
            pltpu.SemaphoreType.DMA,
        ],
        compiler_params=pltpu.CompilerParams(
            collective_id=0, vmem_limit_bytes=60 * 1024 * 1024
        ),
    )(x, W)


# device time: 227634 ns/iter; 1.0750x vs baseline; 1.0750x over previous
import jax
import jax.numpy as jnp
from jax import lax
from jax.experimental import pallas as pl
from jax.experimental.pallas import tpu as pltpu

CHUNK = 1024
N_CHUNKS = 8


def kernel(x, W):
    t, d = x.shape
    _, v_shard = W.shape
    v_total = 2 * v_shard

    def body(x_ref, w_hbm, out_ref, w_buf, copy_sems, send_sems, recv_sems):
        my_x = lax.axis_index("x")
        my_y = lax.axis_index("y")
        my_z = lax.axis_index("z")
        partner = (1 - my_x, my_y, my_z)
        my_off = my_x * v_shard
        opp_off = (1 - my_x) * v_shard

        barrier = pltpu.get_barrier_semaphore()
        pl.semaphore_signal(
            barrier, inc=1, device_id=partner,
            device_id_type=pl.DeviceIdType.MESH,
        )
        pl.semaphore_wait(barrier, 1)

        def chunk_rdma(k):
            return pltpu.make_async_remote_copy(
                src_ref=out_ref.at[:, pl.ds(my_off + k * CHUNK, CHUNK)],
                dst_ref=out_ref.at[:, pl.ds(my_off + k * CHUNK, CHUNK)],
                send_sem=send_sems.at[k],
                recv_sem=recv_sems.at[k],
                device_id=partner,
                device_id_type=pl.DeviceIdType.MESH,
            )

        def w_copy(k, slot):
            return pltpu.make_async_copy(
                w_hbm.at[:, pl.ds(k * CHUNK, CHUNK)],
                w_buf.at[slot],
                copy_sems.at[slot],
            )

        w_copy(0, 0).start()
        xv = x_ref[:, :]
        for k in range(N_CHUNKS):
            slot = k % 2
            if k + 1 < N_CHUNKS:
                w_copy(k + 1, (k + 1) % 2).start()
            w_copy(k, slot).wait()
            logits_k = jnp.dot(
                xv, w_buf[slot], preferred_element_type=jnp.float32
            )
            out_ref[:, pl.ds(my_off + k * CHUNK, CHUNK)] = logits_k
            chunk_rdma(k).start()

        ms = []
        ss = []
        for k in range(N_CHUNKS):
            chunk_rdma(k).wait_send()
            for idx, off in enumerate((my_off, opp_off)):
                if idx == 1:
                    chunk_rdma(k).wait_recv()
                c = out_ref[:, pl.ds(off + k * CHUNK, CHUNK)]
                m_k = jnp.max(c, axis=-1, keepdims=True)
                e = jnp.exp(c - m_k)
                out_ref[:, pl.ds(off + k * CHUNK, CHUNK)] = e
                ms.append((off + k * CHUNK, m_k))
                ss.append(jnp.sum(e, axis=-1, keepdims=True))

        m_g = ms[0][1]
        for _, m_k in ms[1:]:
            m_g = jnp.maximum(m_g, m_k)
        s_g = ss[0] * jnp.exp(ms[0][1] - m_g)
        for (_, m_k), s_k in zip(ms[1:], ss[1:]):
            s_g = s_g + s_k * jnp.exp(m_k - m_g)
        inv = 1.0 / s_g
        for off, m_k in ms:
            scale = jnp.exp(m_k - m_g) * inv
            out_ref[:, pl.ds(off, CHUNK)] = (
                out_ref[:, pl.ds(off, CHUNK)] * scale
            )

    return pl.pallas_call(
        body,
        out_shape=jax.ShapeDtypeStruct((t, v_total), jnp.float32),
        in_specs=[
            pl.BlockSpec(memory_space=pltpu.VMEM),
            pl.BlockSpec(memory_space=pl.ANY),
        ],
        out_specs=pl.BlockSpec(memory_space=pltpu.VMEM),
        scratch_shapes=[
            pltpu.VMEM((2, d, CHUNK), jnp.float32),
            pltpu.SemaphoreType.DMA((2,)),
            pltpu.SemaphoreType.DMA((N_CHUNKS,)),
            pltpu.SemaphoreType.DMA((N_CHUNKS,)),
        ],
        compiler_params=pltpu.CompilerParams(
            collective_id=0, vmem_limit_bytes=60 * 1024 * 1024
        ),
    )(x, W)
